# baseline (speedup 1.0000x reference)
import functools

import jax
import jax.numpy as jnp
from jax import lax
from jax.experimental import pallas as pl
from jax.experimental.pallas import tpu as pltpu
from jax.experimental.pallas import tpu_sc as plsc

E = 8
K = 2
T = 2048
D = 768
F = 768
LANES = 128
NEG = -1e30

B = 256
C = T * K + E * B
NB = C // B
P = T * K

NW = 32
PPW = P // NW
VPW = PPW // 16
TPW = T // NW


def _gating_body(x_ref, gwp_ref, gbp_ref, smask_ref, expand_ref, tri_ref,
                 tle_ref, topv_ref, slot_ref, be_ref):
    logits = jnp.dot(x_ref[...], gwp_ref[...], preferred_element_type=jnp.float32)
    logits = logits + gbp_ref[...]
    m = jnp.max(logits, axis=-1, keepdims=True)
    p = jnp.exp(logits - m)
    gates = p / jnp.sum(p, axis=-1, keepdims=True)
    iota = lax.broadcasted_iota(jnp.int32, (T, LANES), 1)
    v1 = jnp.max(gates, axis=-1, keepdims=True)
    i1 = jnp.min(jnp.where(gates == v1, iota, LANES), axis=-1, keepdims=True)
    g2 = jnp.where(iota == i1, NEG, gates)
    v2 = jnp.max(g2, axis=-1, keepdims=True)
    i2 = jnp.min(jnp.where(g2 == v2, iota, LANES), axis=-1, keepdims=True)
    s = v1 + v2
    topv_ref[...] = jnp.concatenate([v1 / s, v2 / s], axis=1)

    io16 = lax.broadcasted_iota(jnp.int32, (T, 16), 1)
    oh = (jnp.where(io16 == i1, 1.0, 0.0) + jnp.where(io16 == i2, 1.0, 0.0))

    pref = jnp.dot(smask_ref[...], oh, precision=lax.Precision.HIGHEST,
                   preferred_element_type=jnp.float32)
    tot = pref[NW:NW + 1, :]
    padded = jnp.ceil(tot * (1.0 / B)) * B
    off = jnp.dot(padded, tle_ref[...], precision=lax.Precision.HIGHEST,
                  preferred_element_type=jnp.float32)
    prefw = jnp.dot(expand_ref[...], pref, precision=lax.Precision.HIGHEST,
                    preferred_element_type=jnp.float32)

    blks = [jnp.dot(tri_ref[...], oh[b * TPW:(b + 1) * TPW, :],
                    precision=lax.Precision.HIGHEST,
                    preferred_element_type=jnp.float32) for b in range(NW)]
    within = jnp.concatenate(blks, axis=0)

    slotv = off + prefw + within
    s1 = jnp.sum(jnp.where(io16 == i1, slotv, 0.0), axis=-1, keepdims=True)
    s2 = jnp.sum(jnp.where(io16 == i2, slotv, 0.0), axis=-1, keepdims=True)
    slot_ref[...] = jnp.concatenate([s1, s2], axis=1).astype(jnp.int32)

    end = off + padded
    blkrow = lax.broadcasted_iota(jnp.int32, (32, 1), 0).astype(jnp.float32) * B
    lanemask = jnp.where(lax.broadcasted_iota(jnp.int32, (32, 16), 1) < E,
                         1.0, 0.0)
    eb = jnp.sum(jnp.where(blkrow >= end, lanemask, 0.0), axis=-1,
                 keepdims=True)
    be_ref[...] = jnp.minimum(eb, E - 1).astype(jnp.int32)


def _gating(x, gate_w, gate_b):
    gwp = jnp.zeros((D, LANES), jnp.float32).at[:, :E].set(gate_w)
    gbp = jnp.full((LANES,), NEG, jnp.float32).at[:E].set(gate_b)
    row = jnp.arange(40, dtype=jnp.int32)[:, None]
    col = jnp.arange(T, dtype=jnp.int32)[None, :]
    smask = jnp.where((col < row * TPW) | (row == NW), 1.0, 0.0)
    tok = jnp.arange(T, dtype=jnp.int32)[:, None]
    wcol = jnp.arange(40, dtype=jnp.int32)[None, :]
    expand = jnp.where(wcol == tok // TPW, 1.0, 0.0)
    q = jnp.arange(TPW, dtype=jnp.int32)
    tri = jnp.where(q[None, :] < q[:, None], 1.0, 0.0)
    e16 = jnp.arange(16, dtype=jnp.int32)
    tle = jnp.where(e16[:, None] < e16[None, :], 1.0, 0.0)
    return pl.pallas_call(
        _gating_body,
        out_shape=(jax.ShapeDtypeStruct((T, K), jnp.float32),
                   jax.ShapeDtypeStruct((T, K), jnp.int32),
                   jax.ShapeDtypeStruct((32, 1), jnp.int32)),
    )(x, gwp, gbp, smask, expand, tri, tle)


def _dispatch_body(gv_hbm, x_hbm, slot_hbm,
                   gatev_hbm, xs_hbm,
                   slotbuf, tokbuf, gvbuf, rows, sem, sem2):
    wid = lax.axis_index("s") * 2 + lax.axis_index("c")
    base = wid * PPW
    lanes = lax.broadcasted_iota(jnp.int32, (16,), 0)

    pltpu.sync_copy(slot_hbm.at[pl.ds(base, PPW)], slotbuf.at[0])
    pltpu.sync_copy(gv_hbm.at[pl.ds(base, PPW)], gvbuf)
    for j in range(VPW):
        tokbuf[pl.ds(j * 16, 16)] = lax.shift_right_logical(
            base + j * 16 + lanes, 1)

    pltpu.async_copy(gvbuf, gatev_hbm.at[slotbuf.at[0]], sem2).wait()
    pltpu.async_copy(x_hbm.at[tokbuf], rows, sem).wait()
    pltpu.async_copy(rows, xs_hbm.at[slotbuf.at[0]], sem).wait()


def _dispatch(gv_flat, x, slot_flat):
    mesh = plsc.VectorSubcoreMesh(core_axis_name="c", subcore_axis_name="s")
    f = functools.partial(
        pl.kernel,
        mesh=mesh,
        out_type=(jax.ShapeDtypeStruct((C,), jnp.float32),
                  jax.ShapeDtypeStruct((C, D), jnp.float32)),
        scratch_types=[
            pltpu.VMEM((1, PPW), jnp.int32),
            pltpu.VMEM((PPW,), jnp.int32),
            pltpu.VMEM((PPW,), jnp.float32),
            pltpu.VMEM((PPW, D), jnp.float32),
            pltpu.SemaphoreType.DMA,
            pltpu.SemaphoreType.DMA,
        ],
    )(_dispatch_body)
    return f(gv_flat, x, slot_flat)


def _ffn_body(be_ref, xs_ref, w1_ref, b1_ref, w2_ref, b2_ref, gv_ref, y_ref):
    h = jnp.dot(xs_ref[...], w1_ref[0], preferred_element_type=jnp.float32) + b1_ref[0]
    h = jnp.maximum(h, 0.0)
    y = jnp.dot(h, w2_ref[0], preferred_element_type=jnp.float32) + b2_ref[0]
    y_ref[...] = y * gv_ref[...]


def _ffn(xs, gatev, block_e, w1, b1, b2, w2):
    grid_spec = pltpu.PrefetchScalarGridSpec(
        num_scalar_prefetch=1,
        grid=(NB,),
        in_specs=[
            pl.BlockSpec((B, D), lambda i, be: (i, 0)),
            pl.BlockSpec((1, D, F), lambda i, be: (be[i], 0, 0)),
            pl.BlockSpec((1, 1, F), lambda i, be: (be[i], 0, 0)),
            pl.BlockSpec((1, F, D), lambda i, be: (be[i], 0, 0)),
            pl.BlockSpec((1, 1, D), lambda i, be: (be[i], 0, 0)),
            pl.BlockSpec((B, 1), lambda i, be: (i, 0)),
        ],
        out_specs=pl.BlockSpec((B, D), lambda i, be: (i, 0)),
    )
    return pl.pallas_call(
        _ffn_body,
        grid_spec=grid_spec,
        out_shape=jax.ShapeDtypeStruct((C, D), jnp.float32),
    )(block_e, xs, w1, b1.reshape(E, 1, F), w2, b2.reshape(E, 1, D),
      gatev.reshape(C, 1))


HALF = TPW // 2


def _combine_body(slotmap_hbm, y_hbm, out_hbm, slotA, slotB, rowsA, rowsB,
                  obuf, sem):
    wid = lax.axis_index("s") * 2 + lax.axis_index("c")
    tbase = wid * TPW
    for half, slots, rows in ((0, slotA, rowsA), (1, slotB, rowsB)):
        pbase = (tbase + half * HALF) * K
        pltpu.sync_copy(slotmap_hbm.at[pl.ds(pbase, HALF * K)], slots)
        pltpu.async_copy(y_hbm.at[slots], rows, sem).wait()

        def tok(i, _):
            for c in range(D // 16):
                sl = pl.ds(c * 16, 16)
                obuf[i, sl] = rows[2 * i, sl] + rows[2 * i + 1, sl]
            return 0

        lax.fori_loop(0, HALF, tok, 0)
        pltpu.sync_copy(obuf, out_hbm.at[pl.ds(tbase + half * HALF, HALF)])


def _combine(slotmap, y):
    mesh = plsc.VectorSubcoreMesh(core_axis_name="c", subcore_axis_name="s")
    f = functools.partial(
        pl.kernel,
        mesh=mesh,
        out_type=jax.ShapeDtypeStruct((T, D), jnp.float32),
        scratch_types=[
            pltpu.VMEM((HALF * K,), jnp.int32),
            pltpu.VMEM((HALF * K,), jnp.int32),
            pltpu.VMEM((HALF * K, D), jnp.float32),
            pltpu.VMEM((HALF * K, D), jnp.float32),
            pltpu.VMEM((HALF, D), jnp.float32),
            pltpu.SemaphoreType.DMA,
        ],
    )(_combine_body)
    return f(slotmap, y)


@jax.jit
def kernel(x, gate_w, gate_b, w1, b1, w2, b2):
    topv, slot2, be = _gating(x, gate_w, gate_b)
    gv_flat = topv.reshape(P)
    slot_flat = slot2.reshape(P)
    gatev, xs = _dispatch(gv_flat, x, slot_flat)
    y = _ffn(xs, gatev, be.reshape(32), w1, b1, b2, w2)
    return _combine(slot_flat, y)


# speedup vs baseline: 1.0103x; 1.0103x over previous
import functools

import jax
import jax.numpy as jnp
from jax import lax
from jax.experimental import pallas as pl
from jax.experimental.pallas import tpu as pltpu
from jax.experimental.pallas import tpu_sc as plsc

E = 8
K = 2
T = 2048
D = 768
F = 768
LANES = 128
NEG = -1e30

B = 256
C = T * K + E * B
NB = C // B
P = T * K

NW = 32
PPW = P // NW
VPW = PPW // 16
TPW = T // NW


def _gating_body(x_ref, gwp_ref, gbp_ref, smask_ref, expand_ref, tri_ref,
                 tle_ref, topv_ref, slot_ref, be_ref):
    logits = jnp.dot(x_ref[...], gwp_ref[...], preferred_element_type=jnp.float32)
    logits = logits + gbp_ref[...]
    m = jnp.max(logits, axis=-1, keepdims=True)
    p = jnp.exp(logits - m)
    gates = p / jnp.sum(p, axis=-1, keepdims=True)
    iota = lax.broadcasted_iota(jnp.int32, (T, LANES), 1)
    v1 = jnp.max(gates, axis=-1, keepdims=True)
    i1 = jnp.min(jnp.where(gates == v1, iota, LANES), axis=-1, keepdims=True)
    g2 = jnp.where(iota == i1, NEG, gates)
    v2 = jnp.max(g2, axis=-1, keepdims=True)
    i2 = jnp.min(jnp.where(g2 == v2, iota, LANES), axis=-1, keepdims=True)
    s = v1 + v2
    topv_ref[...] = jnp.concatenate([v1 / s, v2 / s], axis=1)

    io16 = lax.broadcasted_iota(jnp.int32, (T, 16), 1)
    oh = (jnp.where(io16 == i1, 1.0, 0.0) + jnp.where(io16 == i2, 1.0, 0.0))

    pref = jnp.dot(smask_ref[...], oh, precision=lax.Precision.HIGHEST,
                   preferred_element_type=jnp.float32)
    tot = pref[NW:NW + 1, :]
    padded = jnp.ceil(tot * (1.0 / B)) * B
    off = jnp.dot(padded, tle_ref[...], precision=lax.Precision.HIGHEST,
                  preferred_element_type=jnp.float32)
    prefw = jnp.dot(expand_ref[...], pref, precision=lax.Precision.HIGHEST,
                    preferred_element_type=jnp.float32)

    blks = [jnp.dot(tri_ref[...], oh[b * TPW:(b + 1) * TPW, :],
                    precision=lax.Precision.HIGHEST,
                    preferred_element_type=jnp.float32) for b in range(NW)]
    within = jnp.concatenate(blks, axis=0)

    slotv = off + prefw + within
    s1 = jnp.sum(jnp.where(io16 == i1, slotv, 0.0), axis=-1, keepdims=True)
    s2 = jnp.sum(jnp.where(io16 == i2, slotv, 0.0), axis=-1, keepdims=True)
    slot_ref[...] = jnp.concatenate([s1, s2], axis=1).astype(jnp.int32)

    end = off + padded
    blkrow = lax.broadcasted_iota(jnp.int32, (32, 1), 0).astype(jnp.float32) * B
    lanemask = jnp.where(lax.broadcasted_iota(jnp.int32, (32, 16), 1) < E,
                         1.0, 0.0)
    eb = jnp.sum(jnp.where(blkrow >= end, lanemask, 0.0), axis=-1,
                 keepdims=True)
    be_ref[...] = jnp.minimum(eb, E - 1).astype(jnp.int32)


def _gating(x, gate_w, gate_b):
    gwp = jnp.zeros((D, LANES), jnp.float32).at[:, :E].set(gate_w)
    gbp = jnp.full((LANES,), NEG, jnp.float32).at[:E].set(gate_b)
    row = jnp.arange(40, dtype=jnp.int32)[:, None]
    col = jnp.arange(T, dtype=jnp.int32)[None, :]
    smask = jnp.where((col < row * TPW) | (row == NW), 1.0, 0.0)
    tok = jnp.arange(T, dtype=jnp.int32)[:, None]
    wcol = jnp.arange(40, dtype=jnp.int32)[None, :]
    expand = jnp.where(wcol == tok // TPW, 1.0, 0.0)
    q = jnp.arange(TPW, dtype=jnp.int32)
    tri = jnp.where(q[None, :] < q[:, None], 1.0, 0.0)
    e16 = jnp.arange(16, dtype=jnp.int32)
    tle = jnp.where(e16[:, None] < e16[None, :], 1.0, 0.0)
    return pl.pallas_call(
        _gating_body,
        out_shape=(jax.ShapeDtypeStruct((T, K), jnp.float32),
                   jax.ShapeDtypeStruct((T, K), jnp.int32),
                   jax.ShapeDtypeStruct((32, 1), jnp.int32)),
    )(x, gwp, gbp, smask, expand, tri, tle)


def _dispatch_body(gv_hbm, x_hbm, slot_hbm,
                   gatev_hbm, xs_hbm,
                   slotbuf, tokbuf, gvbuf, rowsA, rowsB, sem, semb, sem2):
    wid = lax.axis_index("s") * 2 + lax.axis_index("c")
    base = wid * PPW
    H = PPW // 2
    lanes = lax.broadcasted_iota(jnp.int32, (16,), 0)

    pltpu.sync_copy(slot_hbm.at[pl.ds(base, H)], slotbuf.at[0])
    pltpu.sync_copy(slot_hbm.at[pl.ds(base + H, H)], slotbuf.at[1])
    pltpu.sync_copy(gv_hbm.at[pl.ds(base, PPW)], gvbuf)
    for j in range(VPW):
        tokbuf[pl.ds(j * 16, 16)] = lax.shift_right_logical(
            base + j * 16 + lanes, 1)

    H2 = PPW // 2
    gv_dma = pltpu.async_copy(gvbuf.at[pl.ds(0, H2)],
                              gatev_hbm.at[slotbuf.at[0]], sem2)
    gv_dmb = pltpu.async_copy(gvbuf.at[pl.ds(H2, H2)],
                              gatev_hbm.at[slotbuf.at[1]], sem2)
    ga = pltpu.async_copy(x_hbm.at[tokbuf.at[pl.ds(0, H)]], rowsA, sem)
    gb = pltpu.async_copy(x_hbm.at[tokbuf.at[pl.ds(H, H)]], rowsB, semb)
    ga.wait()
    pltpu.async_copy(rowsA, xs_hbm.at[slotbuf.at[0]], sem).wait()
    gb.wait()
    pltpu.async_copy(rowsB, xs_hbm.at[slotbuf.at[1]], semb).wait()
    gv_dma.wait()
    gv_dmb.wait()


def _dispatch(gv_flat, x, slot_flat):
    mesh = plsc.VectorSubcoreMesh(core_axis_name="c", subcore_axis_name="s")
    f = functools.partial(
        pl.kernel,
        mesh=mesh,
        out_type=(jax.ShapeDtypeStruct((C,), jnp.float32),
                  jax.ShapeDtypeStruct((C, D), jnp.float32)),
        scratch_types=[
            pltpu.VMEM((2, PPW // 2), jnp.int32),
            pltpu.VMEM((PPW,), jnp.int32),
            pltpu.VMEM((PPW,), jnp.float32),
            pltpu.VMEM((PPW // 2, D), jnp.float32),
            pltpu.VMEM((PPW // 2, D), jnp.float32),
            pltpu.SemaphoreType.DMA,
            pltpu.SemaphoreType.DMA,
            pltpu.SemaphoreType.DMA,
        ],
    )(_dispatch_body)
    return f(gv_flat, x, slot_flat)


def _ffn_body(be_ref, xs_ref, w1_ref, b1_ref, w2_ref, b2_ref, gv_ref, y_ref):
    h = jnp.dot(xs_ref[...], w1_ref[0], preferred_element_type=jnp.float32) + b1_ref[0]
    h = jnp.maximum(h, 0.0)
    y = jnp.dot(h, w2_ref[0], preferred_element_type=jnp.float32) + b2_ref[0]
    y_ref[...] = y * gv_ref[...]


def _ffn(xs, gatev, block_e, w1, b1, b2, w2):
    grid_spec = pltpu.PrefetchScalarGridSpec(
        num_scalar_prefetch=1,
        grid=(NB,),
        in_specs=[
            pl.BlockSpec((B, D), lambda i, be: (i, 0)),
            pl.BlockSpec((1, D, F), lambda i, be: (be[i], 0, 0)),
            pl.BlockSpec((1, 1, F), lambda i, be: (be[i], 0, 0)),
            pl.BlockSpec((1, F, D), lambda i, be: (be[i], 0, 0)),
            pl.BlockSpec((1, 1, D), lambda i, be: (be[i], 0, 0)),
            pl.BlockSpec((B, 1), lambda i, be: (i, 0)),
        ],
        out_specs=pl.BlockSpec((B, D), lambda i, be: (i, 0)),
    )
    return pl.pallas_call(
        _ffn_body,
        grid_spec=grid_spec,
        out_shape=jax.ShapeDtypeStruct((C, D), jnp.float32),
    )(block_e, xs, w1, b1.reshape(E, 1, F), w2, b2.reshape(E, 1, D),
      gatev.reshape(C, 1))


HALF = TPW // 2


def _combine_body(slotmap_hbm, y_hbm, out_hbm, slotA, slotB, rowsA, rowsB,
                  obuf, semA, semB):
    wid = lax.axis_index("s") * 2 + lax.axis_index("c")
    tbase = wid * TPW
    pltpu.sync_copy(slotmap_hbm.at[pl.ds(tbase * K, HALF * K)], slotA)
    pltpu.sync_copy(slotmap_hbm.at[pl.ds((tbase + HALF) * K, HALF * K)], slotB)
    dA = pltpu.async_copy(y_hbm.at[slotA], rowsA, semA)
    dB = pltpu.async_copy(y_hbm.at[slotB], rowsB, semB)
    for half, rows, dma in ((0, rowsA, dA), (1, rowsB, dB)):
        dma.wait()

        def tok(i, _):
            for c in range(D // 16):
                sl = pl.ds(c * 16, 16)
            return 0

        def tok2(i, _):
            for c in range(D // 16):
                sl = pl.ds(c * 16, 16)
                obuf[i, sl] = rows[2 * i, sl] + rows[2 * i + 1, sl]
            return 0

        lax.fori_loop(0, HALF, tok2, 0)
        pltpu.sync_copy(obuf, out_hbm.at[pl.ds(tbase + half * HALF, HALF)])


def _combine(slotmap, y):
    mesh = plsc.VectorSubcoreMesh(core_axis_name="c", subcore_axis_name="s")
    f = functools.partial(
        pl.kernel,
        mesh=mesh,
        out_type=jax.ShapeDtypeStruct((T, D), jnp.float32),
        scratch_types=[
            pltpu.VMEM((HALF * K,), jnp.int32),
            pltpu.VMEM((HALF * K,), jnp.int32),
            pltpu.VMEM((HALF * K, D), jnp.float32),
            pltpu.VMEM((HALF * K, D), jnp.float32),
            pltpu.VMEM((HALF, D), jnp.float32),
            pltpu.SemaphoreType.DMA,
            pltpu.SemaphoreType.DMA,
        ],
    )(_combine_body)
    return f(slotmap, y)


@jax.jit
def kernel(x, gate_w, gate_b, w1, b1, w2, b2):
    topv, slot2, be = _gating(x, gate_w, gate_b)
    gv_flat = topv.reshape(P)
    slot_flat = slot2.reshape(P)
    gatev, xs = _dispatch(gv_flat, x, slot_flat)
    y = _ffn(xs, gatev, be.reshape(32), w1, b1, b2, w2)
    return _combine(slot_flat, y)
